# baseline (speedup 1.0000x reference)
import jax
import jax.numpy as jnp
from jax import lax
from jax.experimental import pallas as pl
from jax.experimental.pallas import tpu as pltpu
from jax.experimental.pallas import tpu_sc as plsc

L, B, T, H, K, N, R = 4, 8, 128, 512, 4, 2048, 128
NC, NS = 2, 16
NW = NC * NS
BT = B * T
PW = BT // NW


HK = PW * K // 2


def _sc_gather_body(heads_hbm, tails_hbm, relidx_hbm, se_hbm, rel_table_hbm,
                    he_out, te_out, rel_out,
                    ilo, ihi, idxr_v, rows_a, rows_b, acc1, acc2, relrows_v,
                    sem1, sem2, sem3, sem4):
    wid = lax.axis_index("s") * NC + lax.axis_index("c")
    base = wid * PW
    off = (wid // (T // PW)) * N

    def load_idx(src_idx_hbm, dst, half):
        pltpu.sync_copy(src_idx_hbm.at[pl.ds(base * K + half * HK, HK)], dst)
        for i in range(HK // 16):
            dst[pl.ds(i * 16, 16)] = dst[pl.ds(i * 16, 16)] + off

    def reduce_half(rows, acc, half):
        def red(i, c):
            for hh in range(H // 16):
                s = pl.ds(hh * 16, 16)
                acc[half * (PW // 2) + i, s] = (
                    rows[4 * i, s] + rows[4 * i + 1, s]
                    + rows[4 * i + 2, s] + rows[4 * i + 3, s])
            return c
        lax.fori_loop(0, PW // 2, red, 0)

    load_idx(heads_hbm, ilo, 0)
    cpa = pltpu.async_copy(se_hbm.at[ilo], rows_a, sem1)
    load_idx(heads_hbm, ihi, 1)
    cpb = pltpu.async_copy(se_hbm.at[ihi], rows_b, sem2)
    pltpu.sync_copy(relidx_hbm.at[pl.ds(base, PW)], idxr_v)
    cpr = pltpu.async_copy(rel_table_hbm.at[idxr_v], relrows_v, sem4)

    cpa.wait()
    reduce_half(rows_a, acc1, 0)
    load_idx(tails_hbm, ilo, 0)
    cpa2 = pltpu.async_copy(se_hbm.at[ilo], rows_a, sem1)
    cpb.wait()
    reduce_half(rows_b, acc1, 1)
    he_cp = pltpu.async_copy(acc1, he_out.at[pl.ds(base, PW)], sem3)
    load_idx(tails_hbm, ihi, 1)
    cpb2 = pltpu.async_copy(se_hbm.at[ihi], rows_b, sem2)

    cpa2.wait()
    reduce_half(rows_a, acc2, 0)
    cpb2.wait()
    reduce_half(rows_b, acc2, 1)
    he_cp.wait()
    pltpu.sync_copy(acc2, te_out.at[pl.ds(base, PW)])
    cpr.wait()
    pltpu.sync_copy(relrows_v, rel_out.at[pl.ds(base, PW)])


def _main_body(h_ref, he_ref, te_ref, rel_ref, mask_ref,
               w1_ref, w2_ref, bsq_ref, wq_ref, bq_ref, wk_ref, bk_ref,
               wd_ref, bd_ref, wg_ref, bg_ref, wa_ref, ba_ref, wu_ref, bu_ref,
               out_ref, u_scr, wd_scr, ug_scr, pasi_scr, ewu_scr):
    h = h_ref[...]
    rel = rel_ref[...]
    m = mask_ref[...][:, 0, :]

    scale = 1.0 / (H ** 0.5)

    def mm(x, w):
        return jax.lax.dot_general(
            x, w, (((2,), (0,)), ((), ())),
            preferred_element_type=jnp.float32)

    q = mm(h, wq_ref[...]) + bq_ref[...]
    kk = mm(rel, wk_ref[...]) + bk_ref[...]
    scores = jax.lax.dot_general(
        q, kk, (((2,), (2,)), ((0,), (0,))),
        preferred_element_type=jnp.float32) * scale
    neg = jnp.where(m == 1, 0.0, -jnp.inf)
    scores = scores + neg[:, None, :]
    smax = jnp.max(scores, axis=-1, keepdims=True)
    e = jnp.exp(scores - smax)
    pasi_scr[...] = e / jnp.sum(e, axis=-1, keepdims=True)

    log2e = 1.4426950408889634
    wd = mm(h, wd_ref[...]) + bd_ref[...]
    ug_scr[...] = mm(h, wg_ref[...]) + bg_ref[...]

    u0 = mm(he_ref[...], w1_ref[...]) + mm(te_ref[...], w2_ref[...]) \
        + bsq_ref[...]
    mf = (m == 1).astype(jnp.float32)
    u_scr[...] = u0 * mf[:, :, None]

    h_last = h[:, T - 1:T, :]
    a_last = jax.nn.sigmoid(
        jax.lax.dot_general(h_last, wa_ref[...], (((2,), (0,)), ((), ())),
                            preferred_element_type=jnp.float32)
        + ba_ref[...])
    inva = 1.0 / a_last
    wd_scr[...] = inva * jnp.exp2((wd + bu_ref[...]) * (-log2e))

    wu_vec = wu_ref[...] * (-log2e)

    def contract(tp, u):
        p = pasi_scr[:, pl.ds(tp, 1), :]
        out_ref[:, pl.ds(tp, 1), :] = jax.lax.dot_general(
            p, u, (((2,), (1,)), ((0,), (0,))),
            preferred_element_type=jnp.float32)

    def matvec(x):
        return jax.lax.dot_general(x, wu_vec, (((2,), (0,)), ((), ())),
                                   preferred_element_type=jnp.float32)

    def step(t, c):
        u = u_scr[...]
        ewu = ewu_scr[...]
        contract(jnp.maximum(t - 1, 0), u)
        ent = wd_scr[:, pl.ds(t, 1), :]
        ugt = ug_scr[:, pl.ds(t, 1), :]
        bta = 1.0 / (inva + ent * ewu)
        un = u + bta * (ugt - u)
        u_scr[...] = un
        ewu_scr[...] = jnp.exp2(matvec(un))
        return c

    ewu_scr[...] = jnp.exp2(matvec(u_scr[...]))
    jax.lax.fori_loop(0, T, step, 0)
    contract(T - 1, u_scr[...])


def kernel(batched_hidden_states, heads, tails, tri_mask, relations_idx,
           student_embeddings, rel_table, W_sq, b_sq, W_a, b_a, Wq, bq,
           Wk, bk, Wd, bd, Wu, bu, Wg, bg):
    h_last = batched_hidden_states[L - 1]
    mask3 = tri_mask.reshape(B, 1, T).astype(jnp.int32)
    heads_flat = heads.astype(jnp.int32).reshape(BT * K)
    tails_flat = tails.astype(jnp.int32).reshape(BT * K)
    relidx_flat = relations_idx.astype(jnp.int32).reshape(BT)
    se_flat = student_embeddings.reshape(B * N, H)

    sc_gather = pl.kernel(
        _sc_gather_body,
        mesh=plsc.VectorSubcoreMesh(core_axis_name="c", subcore_axis_name="s"),
        out_type=[jax.ShapeDtypeStruct((BT, H), jnp.float32)] * 3,
        scratch_types=[
            pltpu.VMEM((HK,), jnp.int32),
            pltpu.VMEM((HK,), jnp.int32),
            pltpu.VMEM((PW,), jnp.int32),
            pltpu.VMEM((HK, H), jnp.float32),
            pltpu.VMEM((HK, H), jnp.float32),
            pltpu.VMEM((PW, H), jnp.float32),
            pltpu.VMEM((PW, H), jnp.float32),
            pltpu.VMEM((PW, H), jnp.float32),
            pltpu.SemaphoreType.DMA,
            pltpu.SemaphoreType.DMA,
            pltpu.SemaphoreType.DMA,
            pltpu.SemaphoreType.DMA,
        ],
    )
    he, te, rel = sc_gather(heads_flat, tails_flat, relidx_flat,
                            se_flat, rel_table)
    he = he.reshape(B, T, H)
    te = te.reshape(B, T, H)
    rel = rel.reshape(B, T, H)

    w1 = W_sq[:, :H].T
    w2 = W_sq[:, H:].T

    out = pl.pallas_call(
        _main_body,
        in_specs=[
            pl.BlockSpec((B, T, H), lambda: (0, 0, 0)),
            pl.BlockSpec((B, T, H), lambda: (0, 0, 0)),
            pl.BlockSpec((B, T, H), lambda: (0, 0, 0)),
            pl.BlockSpec((B, T, H), lambda: (0, 0, 0)),
            pl.BlockSpec((B, 1, T), lambda: (0, 0, 0)),
            pl.BlockSpec((H, H), lambda: (0, 0)),
            pl.BlockSpec((H, H), lambda: (0, 0)),
            pl.BlockSpec((1, H), lambda: (0, 0)),
            pl.BlockSpec((H, H), lambda: (0, 0)),
            pl.BlockSpec((1, H), lambda: (0, 0)),
            pl.BlockSpec((H, H), lambda: (0, 0)),
            pl.BlockSpec((1, H), lambda: (0, 0)),
            pl.BlockSpec((H, H), lambda: (0, 0)),
            pl.BlockSpec((1, H), lambda: (0, 0)),
            pl.BlockSpec((H, H), lambda: (0, 0)),
            pl.BlockSpec((1, H), lambda: (0, 0)),
            pl.BlockSpec((H, 1), lambda: (0, 0)),
            pl.BlockSpec((1, 1), lambda: (0, 0)),
            pl.BlockSpec((H, 1), lambda: (0, 0)),
            pl.BlockSpec((1, 1), lambda: (0, 0)),
        ],
        out_specs=pl.BlockSpec((B, T, H), lambda: (0, 0, 0)),
        out_shape=jax.ShapeDtypeStruct((B, T, H), jnp.float32),
        scratch_shapes=[
            pltpu.VMEM((B, T, H), jnp.float32),
            pltpu.VMEM((B, T, H), jnp.float32),
            pltpu.VMEM((B, T, H), jnp.float32),
            pltpu.VMEM((B, T, T), jnp.float32),
            pltpu.VMEM((B, T, 1), jnp.float32),
        ],
    )(h_last, he, te, rel, mask3,
      w1, w2, b_sq.reshape(1, H), Wq.T, bq.reshape(1, H), Wk.T,
      bk.reshape(1, H), Wd.T, bd.reshape(1, H), Wg.T, bg.reshape(1, H),
      W_a.T, b_a.reshape(1, 1), Wu.T, bu.reshape(1, 1))
    return out


# speedup vs baseline: 1.0085x; 1.0085x over previous
import jax
import jax.numpy as jnp
from jax import lax
from jax.experimental import pallas as pl
from jax.experimental.pallas import tpu as pltpu
from jax.experimental.pallas import tpu_sc as plsc

L, B, T, H, K, N, R = 4, 8, 128, 512, 4, 2048, 128
NC, NS = 2, 16
NW = NC * NS
BT = B * T
PW = BT // NW


NQ = 4
QK = PW * K // NQ
QP = PW // NQ


def _sc_gather_body(heads_hbm, tails_hbm, relidx_hbm, se_hbm, rel_table_hbm,
                    he_out, te_out, rel_out,
                    idxh, idxt, idxr_v, r0, r1, r2, r3, acc1, acc2, relrows_v,
                    s0, s1, s2, s3, sw, sr):
    wid = lax.axis_index("s") * NC + lax.axis_index("c")
    base = wid * PW
    off = (wid // (T // PW)) * N
    bufs = (r0, r1, r2, r3)
    sems = (s0, s1, s2, s3)

    def load_idx(src_idx_hbm, dst):
        pltpu.sync_copy(src_idx_hbm.at[pl.ds(base * K, PW * K)], dst)
        for i in range(PW * K // 16):
            dst[pl.ds(i * 16, 16)] = dst[pl.ds(i * 16, 16)] + off

    def gather_q(idx, q, c):
        return pltpu.async_copy(se_hbm.at[idx.at[pl.ds(q * QK, QK)]],
                                bufs[c], sems[c])

    def reduce_q(rows, acc, q):
        def red(i, c):
            for hh in range(H // 16):
                s = pl.ds(hh * 16, 16)
                acc[q * QP + i, s] = (
                    rows[4 * i, s] + rows[4 * i + 1, s]
                    + rows[4 * i + 2, s] + rows[4 * i + 3, s])
            return c
        lax.fori_loop(0, QP, red, 0)

    load_idx(heads_hbm, idxh)
    cps = [gather_q(idxh, q, q) for q in range(NQ)]
    load_idx(tails_hbm, idxt)
    pltpu.sync_copy(relidx_hbm.at[pl.ds(base, PW)], idxr_v)
    cpr = pltpu.async_copy(rel_table_hbm.at[idxr_v], relrows_v, sr)

    cps2 = []
    for q in range(NQ):
        cps[q].wait()
        reduce_q(bufs[q], acc1, q)
        cps2.append(gather_q(idxt, q, q))
    he_cp = pltpu.async_copy(acc1, he_out.at[pl.ds(base, PW)], sw)
    for q in range(NQ):
        cps2[q].wait()
        reduce_q(bufs[q], acc2, q)
    he_cp.wait()
    pltpu.sync_copy(acc2, te_out.at[pl.ds(base, PW)])
    cpr.wait()
    pltpu.sync_copy(relrows_v, rel_out.at[pl.ds(base, PW)])


def _main_body(h_ref, he_ref, te_ref, rel_ref, mask_ref,
               w1_ref, w2_ref, bsq_ref, wq_ref, bq_ref, wk_ref, bk_ref,
               wd_ref, bd_ref, wg_ref, bg_ref, wa_ref, ba_ref, wu_ref, bu_ref,
               out_ref, u_scr, wd_scr, ug_scr, pasi_scr, ewu_scr):
    h = h_ref[...]
    rel = rel_ref[...]
    m = mask_ref[...][:, 0, :]

    scale = 1.0 / (H ** 0.5)

    def mm(x, w):
        return jax.lax.dot_general(
            x, w, (((2,), (0,)), ((), ())),
            preferred_element_type=jnp.float32)

    q = mm(h, wq_ref[...]) + bq_ref[...]
    kk = mm(rel, wk_ref[...]) + bk_ref[...]
    scores = jax.lax.dot_general(
        q, kk, (((2,), (2,)), ((0,), (0,))),
        preferred_element_type=jnp.float32) * scale
    neg = jnp.where(m == 1, 0.0, -jnp.inf)
    scores = scores + neg[:, None, :]
    smax = jnp.max(scores, axis=-1, keepdims=True)
    e = jnp.exp(scores - smax)
    pasi_scr[...] = e / jnp.sum(e, axis=-1, keepdims=True)

    log2e = 1.4426950408889634
    wd = mm(h, wd_ref[...]) + bd_ref[...]
    ug_scr[...] = mm(h, wg_ref[...]) + bg_ref[...]

    u0 = mm(he_ref[...], w1_ref[...]) + mm(te_ref[...], w2_ref[...]) \
        + bsq_ref[...]
    mf = (m == 1).astype(jnp.float32)
    u_scr[...] = u0 * mf[:, :, None]

    h_last = h[:, T - 1:T, :]
    a_last = jax.nn.sigmoid(
        jax.lax.dot_general(h_last, wa_ref[...], (((2,), (0,)), ((), ())),
                            preferred_element_type=jnp.float32)
        + ba_ref[...])
    inva = 1.0 / a_last
    wd_scr[...] = inva * jnp.exp2((wd + bu_ref[...]) * (-log2e))

    wu_vec = wu_ref[...] * (-log2e)

    def contract(tp, u):
        p = pasi_scr[:, pl.ds(tp, 1), :]
        out_ref[:, pl.ds(tp, 1), :] = jax.lax.dot_general(
            p, u, (((2,), (1,)), ((0,), (0,))),
            preferred_element_type=jnp.float32)

    def matvec(x):
        return jax.lax.dot_general(x, wu_vec, (((2,), (0,)), ((), ())),
                                   preferred_element_type=jnp.float32)

    def step(t, c):
        u = u_scr[...]
        ewu = ewu_scr[...]
        contract(jnp.maximum(t - 1, 0), u)
        ent = wd_scr[:, pl.ds(t, 1), :]
        ugt = ug_scr[:, pl.ds(t, 1), :]
        bta = 1.0 / (inva + ent * ewu)
        un = u + bta * (ugt - u)
        u_scr[...] = un
        ewu_scr[...] = jnp.exp2(matvec(un))
        return c

    ewu_scr[...] = jnp.exp2(matvec(u_scr[...]))
    jax.lax.fori_loop(0, T, step, 0)
    contract(T - 1, u_scr[...])


def kernel(batched_hidden_states, heads, tails, tri_mask, relations_idx,
           student_embeddings, rel_table, W_sq, b_sq, W_a, b_a, Wq, bq,
           Wk, bk, Wd, bd, Wu, bu, Wg, bg):
    h_last = batched_hidden_states[L - 1]
    mask3 = tri_mask.reshape(B, 1, T).astype(jnp.int32)
    heads_flat = heads.astype(jnp.int32).reshape(BT * K)
    tails_flat = tails.astype(jnp.int32).reshape(BT * K)
    relidx_flat = relations_idx.astype(jnp.int32).reshape(BT)
    se_flat = student_embeddings.reshape(B * N, H)

    sc_gather = pl.kernel(
        _sc_gather_body,
        mesh=plsc.VectorSubcoreMesh(core_axis_name="c", subcore_axis_name="s"),
        out_type=[jax.ShapeDtypeStruct((BT, H), jnp.float32)] * 3,
        scratch_types=[
            pltpu.VMEM((PW * K,), jnp.int32),
            pltpu.VMEM((PW * K,), jnp.int32),
            pltpu.VMEM((PW,), jnp.int32),
            pltpu.VMEM((QK, H), jnp.float32),
            pltpu.VMEM((QK, H), jnp.float32),
            pltpu.VMEM((QK, H), jnp.float32),
            pltpu.VMEM((QK, H), jnp.float32),
            pltpu.VMEM((PW, H), jnp.float32),
            pltpu.VMEM((PW, H), jnp.float32),
            pltpu.VMEM((PW, H), jnp.float32),
            pltpu.SemaphoreType.DMA,
            pltpu.SemaphoreType.DMA,
            pltpu.SemaphoreType.DMA,
            pltpu.SemaphoreType.DMA,
            pltpu.SemaphoreType.DMA,
            pltpu.SemaphoreType.DMA,
        ],
    )
    he, te, rel = sc_gather(heads_flat, tails_flat, relidx_flat,
                            se_flat, rel_table)
    he = he.reshape(B, T, H)
    te = te.reshape(B, T, H)
    rel = rel.reshape(B, T, H)

    w1 = W_sq[:, :H].T
    w2 = W_sq[:, H:].T

    out = pl.pallas_call(
        _main_body,
        in_specs=[
            pl.BlockSpec((B, T, H), lambda: (0, 0, 0)),
            pl.BlockSpec((B, T, H), lambda: (0, 0, 0)),
            pl.BlockSpec((B, T, H), lambda: (0, 0, 0)),
            pl.BlockSpec((B, T, H), lambda: (0, 0, 0)),
            pl.BlockSpec((B, 1, T), lambda: (0, 0, 0)),
            pl.BlockSpec((H, H), lambda: (0, 0)),
            pl.BlockSpec((H, H), lambda: (0, 0)),
            pl.BlockSpec((1, H), lambda: (0, 0)),
            pl.BlockSpec((H, H), lambda: (0, 0)),
            pl.BlockSpec((1, H), lambda: (0, 0)),
            pl.BlockSpec((H, H), lambda: (0, 0)),
            pl.BlockSpec((1, H), lambda: (0, 0)),
            pl.BlockSpec((H, H), lambda: (0, 0)),
            pl.BlockSpec((1, H), lambda: (0, 0)),
            pl.BlockSpec((H, H), lambda: (0, 0)),
            pl.BlockSpec((1, H), lambda: (0, 0)),
            pl.BlockSpec((H, 1), lambda: (0, 0)),
            pl.BlockSpec((1, 1), lambda: (0, 0)),
            pl.BlockSpec((H, 1), lambda: (0, 0)),
            pl.BlockSpec((1, 1), lambda: (0, 0)),
        ],
        out_specs=pl.BlockSpec((B, T, H), lambda: (0, 0, 0)),
        out_shape=jax.ShapeDtypeStruct((B, T, H), jnp.float32),
        scratch_shapes=[
            pltpu.VMEM((B, T, H), jnp.float32),
            pltpu.VMEM((B, T, H), jnp.float32),
            pltpu.VMEM((B, T, H), jnp.float32),
            pltpu.VMEM((B, T, T), jnp.float32),
            pltpu.VMEM((B, T, 1), jnp.float32),
        ],
    )(h_last, he, te, rel, mask3,
      w1, w2, b_sq.reshape(1, H), Wq.T, bq.reshape(1, H), Wk.T,
      bk.reshape(1, H), Wd.T, bd.reshape(1, H), Wg.T, bg.reshape(1, H),
      W_a.T, b_a.reshape(1, 1), Wu.T, bu.reshape(1, 1))
    return out
